# baseline (speedup 1.0000x reference)
import math

import jax
import jax.numpy as jnp
from jax.experimental import pallas as pl
from jax.experimental.pallas import tpu as pltpu
from jax.experimental.pallas import tpu_sc as plsc

_S = 4096
_H = 1024
_NH = 16
_HD = 64
_T = 16
_DIL = 256
_TILE = 512
_NT = _S // _TILE
_SCALE = 1.0 / math.sqrt(_HD)
_EXP_CLAMP = 85.0


def _attn_weights(qk):
    chunks = [qk[_NH * t:_NH * (t + 1), :] for t in range(_T)]
    m_list = [chunks[0]]
    for t in range(1, _T):
        m_list.append(jnp.maximum(m_list[t - 1], chunks[t]))
    M = m_list[-1]
    E = [jnp.exp(chunks[t] - M) for t in range(_T)]
    C = [E[0]]
    for t in range(1, _T):
        C.append(C[t - 1] + E[t])
    l = [C[t] * jnp.exp(jnp.minimum(M - m_list[t], _EXP_CLAMP))
         for t in range(_T)]
    R = [None] * _T
    R[_T - 1] = 1.0 / l[_T - 1]
    for t in range(_T - 2, -1, -1):
        R[t] = R[t + 1] * l[t]
    return jnp.concatenate([E[t] * R[t] for t in range(_T)], axis=0)


def _body(hm_ref, x_tile_ref, x_any_ref, wq_ref, wk_ref, wv_ref, wo_ref,
          out_ref, xsel_ref, kb_ref, vb_ref, wqb_ref, wob_ref, sems):
    i = pl.program_id(0)

    @pl.when(i == 0)
    def _prologue():
        copies = []
        for t in range(_T):
            idx = hm_ref[_DIL * t]
            cp = pltpu.make_async_copy(
                x_any_ref.at[pl.ds(idx, 1), :],
                xsel_ref.at[pl.ds(t, 1), :],
                sems.at[t])
            cp.start()
            copies.append(cp)
        for cp in copies:
            cp.wait()
        xsel = xsel_ref[...]
        k_sel = jax.lax.dot_general(
            xsel, wk_ref[...], (((1,), (1,)), ((), ())),
            preferred_element_type=jnp.float32)
        v_sel = jax.lax.dot_general(
            xsel, wv_ref[...], (((1,), (1,)), ((), ())),
            preferred_element_type=jnp.float32)
        rsel = (jax.lax.broadcasted_iota(jnp.int32, (_T * _NH, _T), 0) // _NH
                == jax.lax.broadcasted_iota(jnp.int32, (_T * _NH, _T), 1)
                ).astype(jnp.float32)
        kb = jax.lax.dot_general(rsel, k_sel, (((1,), (0,)), ((), ())),
                                 preferred_element_type=jnp.float32)
        vb = jax.lax.dot_general(rsel, v_sel, (((1,), (0,)), ((), ())),
                                 preferred_element_type=jnp.float32)
        rr = jax.lax.broadcasted_iota(jnp.int32, (_T * _NH, _H), 0)
        cc = jax.lax.broadcasted_iota(jnp.int32, (_T * _NH, _H), 1)
        bmask = (rr % _NH) == (cc // _HD)
        kb_ref[...] = jnp.where(bmask, kb * _SCALE, 0.0).astype(jnp.bfloat16)
        vb_ref[...] = jnp.where(bmask, vb, 0.0).astype(jnp.bfloat16)
        wqb_ref[...] = wq_ref[...].astype(jnp.bfloat16)
        wob_ref[...] = wo_ref[...].astype(jnp.bfloat16)

    xb = x_tile_ref[...].astype(jnp.bfloat16)
    q = jax.lax.dot_general(
        xb, wqb_ref[...], (((1,), (1,)), ((), ())),
        preferred_element_type=jnp.float32)
    qk_t = jax.lax.dot_general(
        kb_ref[...], q.astype(jnp.bfloat16), (((1,), (1,)), ((), ())),
        preferred_element_type=jnp.float32)
    wm_t = _attn_weights(qk_t)
    o = jax.lax.dot_general(
        wm_t.astype(jnp.bfloat16), vb_ref[...], (((0,), (0,)), ((), ())),
        preferred_element_type=jnp.float32)
    y = jax.lax.dot_general(
        o.astype(jnp.bfloat16), wob_ref[...], (((1,), (1,)), ((), ())),
        preferred_element_type=jnp.float32)
    out_ref[...] = y


def _build(interpret=False):
    grid_spec = pltpu.PrefetchScalarGridSpec(
        num_scalar_prefetch=1,
        grid=(_NT,),
        in_specs=[
            pl.BlockSpec((_TILE, _H), lambda i, hm: (i, 0)),
            pl.BlockSpec(memory_space=pl.ANY),
            pl.BlockSpec((_H, _H), lambda i, hm: (0, 0)),
            pl.BlockSpec((_H, _H), lambda i, hm: (0, 0)),
            pl.BlockSpec((_H, _H), lambda i, hm: (0, 0)),
            pl.BlockSpec((_H, _H), lambda i, hm: (0, 0)),
        ],
        out_specs=pl.BlockSpec((_TILE, _H), lambda i, hm: (i, 0)),
        scratch_shapes=[
            pltpu.VMEM((_T, _H), jnp.float32),
            pltpu.VMEM((_T * _NH, _H), jnp.bfloat16),
            pltpu.VMEM((_T * _NH, _H), jnp.bfloat16),
            pltpu.VMEM((_H, _H), jnp.bfloat16),
            pltpu.VMEM((_H, _H), jnp.bfloat16),
            pltpu.SemaphoreType.DMA((_T,)),
        ],
    )
    return pl.pallas_call(
        _body,
        grid_spec=grid_spec,
        out_shape=jax.ShapeDtypeStruct((_S, _H), jnp.float32),
        interpret=interpret,
    )


_SC_WIN = 32


_SC_UNITS = 32
_SC_ROWS = _S // _SC_UNITS


def _sc_scatter(y2, hm2):
    @pl.kernel(out_type=jax.ShapeDtypeStruct((_S, _H), jnp.float32),
               mesh=plsc.VectorSubcoreMesh(core_axis_name="core",
                                           subcore_axis_name="subcore"),
               scratch_types=[
                   pltpu.VMEM((1, _S), jnp.int32),
                   pltpu.VMEM((2, _SC_WIN, _H), jnp.float32),
                   pltpu.SemaphoreType.DMA,
                   pltpu.SemaphoreType.DMA((2,)),
               ])
    def scatter_kernel(y_hbm, i_hbm, o_hbm, idx_vmem, buf, sem_i, sem_d):
        core = jax.lax.axis_index("core")
        sub = jax.lax.axis_index("subcore")
        base = (core * 16 + sub) * _SC_ROWS
        pltpu.async_copy(i_hbm, idx_vmem, sem_i).wait()
        n_win = _SC_ROWS // _SC_WIN
        cps = []
        for w in range(n_win):
            cp = pltpu.make_async_copy(
                y_hbm.at[pl.ds(base + w * _SC_WIN, _SC_WIN), :],
                buf.at[w % 2], sem_d.at[w % 2])
            cp.start()
            cps.append(cp)
            if w >= 1:
                cps[w - 1].wait()
                s = base + (w - 1) * _SC_WIN
                pltpu.sync_copy(
                    buf.at[(w - 1) % 2],
                    o_hbm.at[idx_vmem.at[0, pl.ds(s, _SC_WIN)]])
        cps[n_win - 1].wait()
        s = base + (n_win - 1) * _SC_WIN
        pltpu.sync_copy(
            buf.at[(n_win - 1) % 2],
            o_hbm.at[idx_vmem.at[0, pl.ds(s, _SC_WIN)]])

    return scatter_kernel(y2, hm2)


def kernel(x, W_qkv, W_out, hilbert_map):
    b, s, h = x.shape
    x2 = x.reshape(s, h)
    wq = W_qkv[:_H]
    wk = W_qkv[_H:2 * _H]
    wv = W_qkv[2 * _H:]
    hm = hilbert_map.astype(jnp.int32)
    y = _build()(hm, x2, x2, wq, wk, wv, W_out)
    out = _sc_scatter(y, hm.reshape(1, _S))
    return out.reshape(b, s, h)


# speedup vs baseline: 1.1972x; 1.1972x over previous
import math

import jax
import jax.numpy as jnp
from jax.experimental import pallas as pl
from jax.experimental.pallas import tpu as pltpu
from jax.experimental.pallas import tpu_sc as plsc

_S = 4096
_H = 1024
_NH = 16
_HD = 64
_T = 16
_DIL = 256
_TILE = 512
_NT = _S // _TILE
_SCALE = 1.0 / math.sqrt(_HD)
_EXP_CLAMP = 85.0


def _attn_weights(qk):
    chunks = [qk[_NH * t:_NH * (t + 1), :] for t in range(_T)]
    m_list = [chunks[0]]
    for t in range(1, _T):
        m_list.append(jnp.maximum(m_list[t - 1], chunks[t]))
    M = m_list[-1]
    E = [jnp.exp(chunks[t] - M) for t in range(_T)]
    C = [E[0]]
    for t in range(1, _T):
        C.append(C[t - 1] + E[t])
    l = [C[t] * jnp.exp(jnp.minimum(M - m_list[t], _EXP_CLAMP))
         for t in range(_T)]
    R = [None] * _T
    R[_T - 1] = 1.0 / l[_T - 1]
    for t in range(_T - 2, -1, -1):
        R[t] = R[t + 1] * l[t]
    return jnp.concatenate([E[t] * R[t] for t in range(_T)], axis=0)


def _body(hm_ref, x_tile_ref, x_any_ref, wq_ref, wk_ref, wv_ref, wo_ref,
          out_ref, xsel_ref, kq_ref, vo_ref, sems):
    i = pl.program_id(0)

    @pl.when(i == 0)
    def _prologue():
        copies = []
        for t in range(_T):
            idx = hm_ref[_DIL * t]
            cp = pltpu.make_async_copy(
                x_any_ref.at[pl.ds(idx, 1), :],
                xsel_ref.at[pl.ds(t, 1), :],
                sems.at[t])
            cp.start()
            copies.append(cp)
        for cp in copies:
            cp.wait()
        xsel = xsel_ref[...]
        k_sel = jax.lax.dot_general(
            xsel, wk_ref[...], (((1,), (1,)), ((), ())),
            preferred_element_type=jnp.float32)
        v_sel = jax.lax.dot_general(
            xsel, wv_ref[...], (((1,), (1,)), ((), ())),
            preferred_element_type=jnp.float32)
        rsel = (jax.lax.broadcasted_iota(jnp.int32, (_T * _NH, _T), 0) // _NH
                == jax.lax.broadcasted_iota(jnp.int32, (_T * _NH, _T), 1)
                ).astype(jnp.float32)
        kb = jax.lax.dot_general(rsel, k_sel, (((1,), (0,)), ((), ())),
                                 preferred_element_type=jnp.float32)
        vb = jax.lax.dot_general(rsel, v_sel, (((1,), (0,)), ((), ())),
                                 preferred_element_type=jnp.float32)
        rr = jax.lax.broadcasted_iota(jnp.int32, (_T * _NH, _H), 0)
        cc = jax.lax.broadcasted_iota(jnp.int32, (_T * _NH, _H), 1)
        bmask = (rr % _NH) == (cc // _HD)
        kbv = jnp.where(bmask, kb * _SCALE, 0.0)
        vbv = jnp.where(bmask, vb, 0.0)
        kq = jax.lax.dot_general(kbv, wq_ref[...], (((1,), (0,)), ((), ())),
                                 preferred_element_type=jnp.float32)
        vo = jax.lax.dot_general(vbv, wo_ref[...], (((1,), (1,)), ((), ())),
                                 preferred_element_type=jnp.float32)
        kq_ref[...] = kq
        vo_ref[...] = vo

    qk_t = jax.lax.dot_general(
        kq_ref[...], x_tile_ref[...], (((1,), (1,)), ((), ())),
        preferred_element_type=jnp.float32)
    wm_t = _attn_weights(qk_t)
    y = jax.lax.dot_general(
        wm_t, vo_ref[...], (((0,), (0,)), ((), ())),
        preferred_element_type=jnp.float32)
    out_ref[...] = y


def _build(interpret=False):
    grid_spec = pltpu.PrefetchScalarGridSpec(
        num_scalar_prefetch=1,
        grid=(_NT,),
        in_specs=[
            pl.BlockSpec((_TILE, _H), lambda i, hm: (i, 0)),
            pl.BlockSpec(memory_space=pl.ANY),
            pl.BlockSpec((_H, _H), lambda i, hm: (0, 0)),
            pl.BlockSpec((_H, _H), lambda i, hm: (0, 0)),
            pl.BlockSpec((_H, _H), lambda i, hm: (0, 0)),
            pl.BlockSpec((_H, _H), lambda i, hm: (0, 0)),
        ],
        out_specs=pl.BlockSpec((_TILE, _H), lambda i, hm: (i, 0)),
        scratch_shapes=[
            pltpu.VMEM((_T, _H), jnp.float32),
            pltpu.VMEM((_T * _NH, _H), jnp.float32),
            pltpu.VMEM((_T * _NH, _H), jnp.float32),
            pltpu.SemaphoreType.DMA((_T,)),
        ],
    )
    return pl.pallas_call(
        _body,
        grid_spec=grid_spec,
        out_shape=jax.ShapeDtypeStruct((_S, _H), jnp.float32),
        interpret=interpret,
    )


_SC_WIN = 32


_SC_UNITS = 32
_SC_ROWS = _S // _SC_UNITS


def _sc_scatter(y2, hm2):
    @pl.kernel(out_type=jax.ShapeDtypeStruct((_S, _H), jnp.float32),
               mesh=plsc.VectorSubcoreMesh(core_axis_name="core",
                                           subcore_axis_name="subcore"),
               scratch_types=[
                   pltpu.VMEM((1, _S), jnp.int32),
                   pltpu.VMEM((2, _SC_WIN, _H), jnp.float32),
                   pltpu.SemaphoreType.DMA,
                   pltpu.SemaphoreType.DMA((2,)),
               ])
    def scatter_kernel(y_hbm, i_hbm, o_hbm, idx_vmem, buf, sem_i, sem_d):
        core = jax.lax.axis_index("core")
        sub = jax.lax.axis_index("subcore")
        base = (core * 16 + sub) * _SC_ROWS
        pltpu.async_copy(i_hbm, idx_vmem, sem_i).wait()
        n_win = _SC_ROWS // _SC_WIN
        cps = []
        for w in range(n_win):
            cp = pltpu.make_async_copy(
                y_hbm.at[pl.ds(base + w * _SC_WIN, _SC_WIN), :],
                buf.at[w % 2], sem_d.at[w % 2])
            cp.start()
            cps.append(cp)
            if w >= 1:
                cps[w - 1].wait()
                s = base + (w - 1) * _SC_WIN
                pltpu.sync_copy(
                    buf.at[(w - 1) % 2],
                    o_hbm.at[idx_vmem.at[0, pl.ds(s, _SC_WIN)]])
        cps[n_win - 1].wait()
        s = base + (n_win - 1) * _SC_WIN
        pltpu.sync_copy(
            buf.at[(n_win - 1) % 2],
            o_hbm.at[idx_vmem.at[0, pl.ds(s, _SC_WIN)]])

    return scatter_kernel(y2, hm2)


def kernel(x, W_qkv, W_out, hilbert_map):
    b, s, h = x.shape
    x2 = x.reshape(s, h)
    wq = W_qkv[:_H]
    wk = W_qkv[_H:2 * _H]
    wv = W_qkv[2 * _H:]
    hm = hilbert_map.astype(jnp.int32)
    y = _build()(hm, x2, x2, wq, wk, wv, W_out)
    out = _sc_scatter(y, hm.reshape(1, _S))
    return out.reshape(b, s, h)
